# baseline (speedup 1.0000x reference)
import jax
import jax.numpy as jnp
from jax.experimental import pallas as pl
from jax.experimental.pallas import tpu as pltpu

B = 1024
M = 16384
BB = 128
BM = 4096
NJ = M // BM


def _body(fg_ref, w0_ref, w1_ref, r0_ref, r1_ref, r2_ref, r3_ref, pu_ref,
          out_ref):
    ww = 1.0 - (1.0 - w0_ref[...]) * (1.0 - w1_ref[...])
    pu = pu_ref[...]
    usage = pu + (1.0 - pu) * ww
    fg = fg_ref[...]
    phi = 1.0 - fg[:, 0][:, None] * r0_ref[...]
    phi = phi * (1.0 - fg[:, 1][:, None] * r1_ref[...])
    phi = phi * (1.0 - fg[:, 2][:, None] * r2_ref[...])
    phi = phi * (1.0 - fg[:, 3][:, None] * r3_ref[...])
    out_ref[...] = jnp.clip(usage * phi, 0.0, 1.0)


def _plane(k):
    return pl.BlockSpec((BB, BM), lambda i, j, k=k: (i, j + k * NJ))


def kernel(write_weights, free_gate, read_weights, prev_usage):
    ww2 = write_weights.reshape(B, 2 * M)
    rw2 = read_weights.reshape(B, 4 * M)
    grid = (B // BB, NJ)
    return pl.pallas_call(
        _body,
        grid=grid,
        in_specs=[
            pl.BlockSpec((BB, 4), lambda i, j: (i, 0)),
            _plane(0), _plane(1),
            _plane(0), _plane(1), _plane(2), _plane(3),
            pl.BlockSpec((BB, BM), lambda i, j: (i, j)),
        ],
        out_specs=pl.BlockSpec((BB, BM), lambda i, j: (i, j)),
        out_shape=jax.ShapeDtypeStruct((B, M), jnp.float32),
        compiler_params=pltpu.CompilerParams(
            dimension_semantics=("arbitrary", "arbitrary"),
        ),
    )(free_gate, ww2, ww2, rw2, rw2, rw2, rw2, prev_usage)
